# Initial kernel scaffold; baseline (speedup 1.0000x reference)
import jax
import jax.numpy as jnp
from jax.experimental import pallas as pl


def kernel(code, static_mask, numeric_value, time_delta_days, numeric_value_mask, mask, table):
    raise NotImplementedError("write your pallas kernel here")


# speedup vs baseline: 3.9626x; 3.9626x over previous
import functools

import jax
import jax.numpy as jnp
from jax import lax
from jax.experimental import pallas as pl
from jax.experimental.pallas import tpu as pltpu
from jax.experimental.pallas import tpu_sc as plsc

TOKEN_DIM = 64
G = 128
NG = 4
CHUNK = G * NG


@functools.partial(jax.jit, static_argnums=())
def _sc_gather(table, idx2):
    n_rows = idx2.shape[0]
    n_total = n_rows * G
    info = plsc.get_sparse_core_info()
    nw = info.num_cores * info.num_subcores
    per_w_rows = n_rows // nw
    chunks = per_w_rows // NG

    mesh = plsc.VectorSubcoreMesh(core_axis_name="c", subcore_axis_name="s")

    @functools.partial(
        pl.kernel,
        out_type=jax.ShapeDtypeStruct((n_total, TOKEN_DIM), jnp.float32),
        mesh=mesh,
        scratch_types=[
            pltpu.VMEM((NG, G), jnp.int32),
            pltpu.VMEM((CHUNK, TOKEN_DIM), jnp.float32),
            pltpu.SemaphoreType.DMA,
        ],
        compiler_params=pltpu.CompilerParams(use_tc_tiling_on_sc=False),
    )
    def k(table_hbm, idx_hbm, out_hbm, idx_v, rows_v, sem_g):
        wid = lax.axis_index("s") * info.num_cores + lax.axis_index("c")
        row_base = wid * per_w_rows

        def body(c, _):
            row_off = row_base + c * NG
            pltpu.sync_copy(idx_hbm.at[pl.ds(row_off, NG)], idx_v)
            copies = []
            for g in range(NG):
                copies.append(
                    pltpu.async_copy(
                        table_hbm.at[idx_v.at[g]],
                        rows_v.at[pl.ds(g * G, G)],
                        sem_g,
                    )
                )
            for cp in copies:
                cp.wait()
            pltpu.sync_copy(
                rows_v, out_hbm.at[pl.ds(row_off * G, CHUNK)]
            )
            return _

        lax.fori_loop(0, chunks, body, 0)

    return k(table, idx2)


def kernel(code, static_mask, numeric_value, time_delta_days,
           numeric_value_mask, mask, table):
    B, L = code.shape
    idx2 = code.astype(jnp.int32).reshape(B * L // G, G)
    out = _sc_gather(table, idx2)
    return out.reshape(B, L, TOKEN_DIM)
